# baseline (speedup 1.0000x reference)
import numpy as np
import jax
import jax.numpy as jnp
from jax.experimental import pallas as pl

_QUALITIES = (20, 40, 60, 80)

_LUMA = np.array([
    [16, 11, 10, 16, 24, 40, 51, 61],
    [12, 12, 14, 19, 26, 58, 60, 55],
    [14, 13, 16, 24, 40, 57, 69, 56],
    [14, 17, 22, 29, 51, 87, 80, 62],
    [18, 22, 37, 56, 68, 109, 103, 77],
    [24, 35, 55, 64, 81, 104, 113, 92],
    [49, 64, 78, 87, 103, 121, 120, 101],
    [72, 92, 95, 98, 112, 100, 103, 99]], dtype=np.float32)
_CHROMA = np.array([
    [17, 18, 24, 47, 99, 99, 99, 99],
    [18, 21, 26, 66, 99, 99, 99, 99],
    [24, 26, 56, 99, 99, 99, 99, 99],
    [47, 66, 99, 99, 99, 99, 99, 99],
    [99, 99, 99, 99, 99, 99, 99, 99],
    [99, 99, 99, 99, 99, 99, 99, 99],
    [99, 99, 99, 99, 99, 99, 99, 99],
    [99, 99, 99, 99, 99, 99, 99, 99]], dtype=np.float32)


def _qtable(base, q):
    s = 5000.0 / q if q < 50 else 200.0 - 2.0 * q
    t = np.floor((base * s + 50.0) / 100.0)
    return np.clip(t, 1.0, 255.0).astype(np.float32)


def _dct_mat():
    k = np.arange(8)
    n = np.arange(8)
    D = np.cos(np.pi * (2 * n[None, :] + 1) * k[:, None] / 16.0)
    D[0, :] *= 1.0 / np.sqrt(2.0)
    D *= np.sqrt(2.0 / 8.0)
    return D.astype(np.float32)


_SLAB = 512
_MB = 256
_D8 = _dct_mat()
_BD = np.kron(np.eye(_MB // 8, dtype=np.float32), _D8)
_BDT = np.ascontiguousarray(_BD.T)
_EROW = np.kron(np.eye(_SLAB // 8, dtype=np.float32),
                np.ones((8, 1), np.float32))


def _tiled_tables(base, w):
    return np.stack([np.tile(_qtable(base, q), (_MB // 8, w // 8))
                     for q in _QUALITIES])


def _slq_kernel(x_ref, z_ref, bd_ref, bdt_ref, erow_ref, ecol_ref,
                tl_ref, tc_ref, o_ref):
    bd = bd_ref[...]
    bdt = bdt_ref[...]
    w = x_ref.shape[3]
    nh = w // _MB

    z = z_ref[0, 0]
    zf = jnp.dot(erow_ref[...], jnp.dot(z, ecol_ref[...]))

    def qsel(tq, zh):
        m0 = zh < 0.5
        m1 = zh < 1.5
        m2 = zh < 2.5
        return jnp.where(m0, tq[0],
               jnp.where(m1, tq[1],
               jnp.where(m2, tq[2], tq[3])))

    def qsel2(tq_ref):
        tq = tq_ref[...]
        return jnp.concatenate(
            [qsel(tq, zf[k * _MB:(k + 1) * _MB]) for k in range(_SLAB // _MB)],
            axis=0)

    qy = qsel2(tl_ref)
    qc = qsel2(tc_ref)

    x = x_ref[0]
    img = jnp.round(jnp.clip(x, 0.0, 1.0) * 255.0)
    R, G, B = img[0], img[1], img[2]
    Y = 0.299 * R + 0.587 * G + 0.114 * B - 128.0
    Cb = -0.168736 * R - 0.331264 * G + 0.5 * B
    Cr = 0.5 * R - 0.418688 * G - 0.081312 * B

    def hmul(a, m):
        return jnp.concatenate(
            [jnp.dot(a[:, i * _MB:(i + 1) * _MB], m) for i in range(nh)],
            axis=1)

    def vmul(m, a):
        return jnp.concatenate(
            [jnp.dot(m, a[k * _MB:(k + 1) * _MB]) for k in range(_SLAB // _MB)],
            axis=0)

    def comp(ch, qp):
        a = vmul(bd, ch)
        coef = hmul(a, bdt)
        cq = jnp.round(coef / qp) * qp
        a2 = vmul(bdt, cq)
        return hmul(a2, bd)

    Y2 = comp(Y, qy)
    Cb2 = comp(Cb, qc)
    Cr2 = comp(Cr, qc)

    R2 = Y2 + 1.402 * Cr2 + 128.0
    G2 = Y2 - 0.344136 * Cb2 - 0.714136 * Cr2 + 128.0
    B2 = Y2 + 1.772 * Cb2 + 128.0

    def finish(c):
        return jnp.clip(jnp.round(c), 0.0, 255.0) * (1.0 / 255.0)

    o_ref[0, 0] = finish(R2)
    o_ref[0, 1] = finish(G2)
    o_ref[0, 2] = finish(B2)


def kernel(inputs):
    B, H, W, C = inputs.shape
    pn, pm = H // 8, W // 8
    nslab = H // _SLAB

    xt = jnp.transpose(inputs, (0, 3, 1, 2))

    Z = jax.random.randint(jax.random.key(42), (B, pn, pm), 0,
                           len(_QUALITIES))
    Zr = Z.reshape(B, nslab, _SLAB // 8, pm).astype(jnp.float32)

    ecol = np.kron(np.eye(pm, dtype=np.float32), np.ones((1, 8), np.float32))
    tl = _tiled_tables(_LUMA, W)
    tc = _tiled_tables(_CHROMA, W)

    out_t = pl.pallas_call(
        _slq_kernel,
        grid=(B, nslab),
        in_specs=[
            pl.BlockSpec((1, 3, _SLAB, W), lambda b, s: (b, 0, s, 0)),
            pl.BlockSpec((1, 1, _SLAB // 8, pm), lambda b, s: (b, s, 0, 0)),
            pl.BlockSpec((_MB, _MB), lambda b, s: (0, 0)),
            pl.BlockSpec((_MB, _MB), lambda b, s: (0, 0)),
            pl.BlockSpec((_SLAB, _SLAB // 8), lambda b, s: (0, 0)),
            pl.BlockSpec((pm, W), lambda b, s: (0, 0)),
            pl.BlockSpec((4, _MB, W), lambda b, s: (0, 0, 0)),
            pl.BlockSpec((4, _MB, W), lambda b, s: (0, 0, 0)),
        ],
        out_specs=pl.BlockSpec((1, 3, _SLAB, W), lambda b, s: (b, 0, s, 0)),
        out_shape=jax.ShapeDtypeStruct((B, 3, H, W), jnp.float32),
    )(xt, Zr, jnp.asarray(_BD), jnp.asarray(_BDT), jnp.asarray(_EROW),
      jnp.asarray(ecol), jnp.asarray(tl), jnp.asarray(tc))

    return jnp.transpose(out_t, (0, 2, 3, 1))
